# baseline (speedup 1.0000x reference)
import jax
import jax.numpy as jnp
from jax import lax
from jax.experimental import pallas as pl
from jax.experimental.pallas import tpu as pltpu
from jax.experimental.pallas import tpu_sc as plsc

_B = 4096
_C = 64
_K = 256
_KW = _K // 2
_IW = 16
_JW = 32
_JWW = _JW // 2
_NIB = _C // _IW
_NJB = _C // _JW
_NBH = 4
_BH = _B // _NBH
_NB = 16
_NCHUNK = _BH // _NB
_RING = 4
_UNROLL = 8


def _body(in_hbm, tab_hbm, out_hbm, tbuf, inbuf, outbuf, isems, osems):
    c = lax.axis_index("c")
    s = lax.axis_index("s")
    wid = s * 2 + c
    ib = wid % _NIB
    jb = (wid // _NIB) % _NJB
    bh = wid // (_NIB * _NJB)
    i0 = ib * _IW
    j0 = jb * _JW
    b0 = bh * _BH

    def in_copy(ck, slot):
        b = b0 + ck * _NB
        return pltpu.make_async_copy(
            in_hbm.at[pl.ds(b, _NB), pl.ds(j0, _JW), pl.ds(i0, _IW)],
            inbuf.at[slot], isems.at[slot])

    def out_copy(ck, slot):
        b = b0 + ck * _NB
        return pltpu.make_async_copy(
            outbuf.at[slot],
            out_hbm.at[pl.ds(b, _NB), pl.ds(i0, _IW), pl.ds(jb * _JWW, _JWW)],
            osems.at[slot])

    for r in range(_RING):
        in_copy(r, r).start()

    pltpu.sync_copy(tab_hbm.at[pl.ds(i0, _IW), pl.ds(j0, _JW), :], tbuf)

    lanes = jnp.arange(16, dtype=jnp.int32)

    def chunk_body(ck, _):
        slot = ck % _RING
        in_copy(ck, slot).wait()

        @pl.when(ck >= _RING)
        def _drain_out():
            out_copy(ck - _RING, slot).wait()

        @plsc.parallel_loop(0, _NB * _JWW, unroll=_UNROLL)
        def g_body(g):
            bb = g // _JWW
            jp = g % _JWW
            jl0 = 2 * jp
            jl1 = jl0 + 1
            codes0 = inbuf[slot, bb, jl0, :]
            codes1 = inbuf[slot, bb, jl1, :]
            w0 = plsc.load_gather(
                tbuf, [lanes, jnp.full((16,), jl0, jnp.int32), codes0 >> 1])
            w1 = plsc.load_gather(
                tbuf, [lanes, jnp.full((16,), jl1, jnp.int32), codes1 >> 1])
            h0 = (w0 >> ((codes0 & 1) << 4)) & 0xFFFF
            h1 = (w1 >> ((codes1 & 1) << 4)) & 0xFFFF
            plsc.store_scatter(
                outbuf, [jnp.full((16,), slot, jnp.int32),
                         jnp.full((16,), bb, jnp.int32), lanes,
                         jnp.full((16,), jp, jnp.int32)],
                h0 | (h1 << 16))

        out_copy(ck, slot).start()

        @pl.when(ck + _RING < _NCHUNK)
        def _start_next():
            in_copy(ck + _RING, slot).start()

        return _

    lax.fori_loop(0, _NCHUNK, chunk_body, None)
    for r in range(_RING):
        out_copy(_NCHUNK - _RING + r, (_NCHUNK - _RING + r) % _RING).wait()


def kernel(input, white_table):
    tab_packed = jax.lax.bitcast_convert_type(
        white_table.astype(jnp.bfloat16).reshape(_C, _C, _KW, 2), jnp.int32)
    mesh = plsc.VectorSubcoreMesh(
        core_axis_name="c", subcore_axis_name="s", num_cores=2, num_subcores=16)
    f = pl.kernel(
        _body,
        out_type=jax.ShapeDtypeStruct((_B, _C, _C // 2), jnp.int32),
        mesh=mesh,
        scratch_types=[
            pltpu.VMEM((_IW, _JW, _KW), jnp.int32),
            pltpu.VMEM((_RING, _NB, _JW, _IW), jnp.int32),
            pltpu.VMEM((_RING, _NB, _IW, _JWW), jnp.int32),
            pltpu.SemaphoreType.DMA((_RING,)),
            pltpu.SemaphoreType.DMA((_RING,)),
        ],
        compiler_params=pltpu.CompilerParams(
            use_tc_tiling_on_sc=False, needs_layout_passes=False),
    )
    out_packed = f(input, tab_packed)
    out = jax.lax.bitcast_convert_type(out_packed, jnp.bfloat16)
    return out.reshape(_B, _C, _C).astype(jnp.float32)


# speedup vs baseline: 1.6444x; 1.6444x over previous
import jax
import jax.numpy as jnp
from jax import lax
from jax.experimental import pallas as pl
from jax.experimental.pallas import tpu as pltpu
from jax.experimental.pallas import tpu_sc as plsc

_B = 4096
_C = 64
_K = 256
_IW = 16
_JW = 16
_NIB = _C // _IW
_NJB = _C // _JW
_NBH = 2
_BH = _B // _NBH
_NB = 16
_NCHUNK = _BH // _NB
_RING = 4
_UNROLL = 8


def _body(in_hbm, tab_hbm, out_hbm, tbuf, inbuf, outbuf, isems, osems):
    c = lax.axis_index("c")
    s = lax.axis_index("s")
    wid = s * 2 + c
    ib = wid % _NIB
    jb = (wid // _NIB) % _NJB
    bh = wid // (_NIB * _NJB)
    i0 = ib * _IW
    j0 = jb * _JW
    b0 = bh * _BH

    def in_copy(ck, slot):
        b = b0 + ck * _NB
        return pltpu.make_async_copy(
            in_hbm.at[pl.ds(b, _NB), pl.ds(j0, _JW), pl.ds(i0, _IW)],
            inbuf.at[slot], isems.at[slot])

    def out_copy(ck, slot):
        b = b0 + ck * _NB
        return pltpu.make_async_copy(
            outbuf.at[slot],
            out_hbm.at[pl.ds(b, _NB), pl.ds(i0, _IW), pl.ds(j0, _JW)],
            osems.at[slot])

    for r in range(_RING):
        in_copy(r, r).start()

    pltpu.sync_copy(tab_hbm.at[pl.ds(i0, _IW), pl.ds(j0, _JW), :], tbuf)

    lanes = jnp.arange(16, dtype=jnp.int32)

    def chunk_body(ck, _):
        slot = ck % _RING
        in_copy(ck, slot).wait()

        @pl.when(ck >= _RING)
        def _drain_out():
            out_copy(ck - _RING, slot).wait()

        @plsc.parallel_loop(0, _NB * _JW, unroll=_UNROLL)
        def g_body(g):
            bb = g // _JW
            jl = g % _JW
            codes = inbuf[slot, bb, jl, :]
            jv = jnp.full((16,), jl, jnp.int32)
            vals = plsc.load_gather(tbuf, [lanes, jv, codes])
            plsc.store_scatter(
                outbuf, [jnp.full((16,), slot, jnp.int32),
                         jnp.full((16,), bb, jnp.int32), lanes, jv], vals)

        out_copy(ck, slot).start()

        @pl.when(ck + _RING < _NCHUNK)
        def _start_next():
            in_copy(ck + _RING, slot).start()

        return _

    lax.fori_loop(0, _NCHUNK, chunk_body, None)
    for r in range(_RING):
        out_copy(_NCHUNK - _RING + r, (_NCHUNK - _RING + r) % _RING).wait()


def kernel(input, white_table):
    mesh = plsc.VectorSubcoreMesh(
        core_axis_name="c", subcore_axis_name="s", num_cores=2, num_subcores=16)
    f = pl.kernel(
        _body,
        out_type=jax.ShapeDtypeStruct((_B, _C, _C), jnp.float32),
        mesh=mesh,
        scratch_types=[
            pltpu.VMEM((_IW, _JW, _K), jnp.float32),
            pltpu.VMEM((_RING, _NB, _JW, _IW), jnp.int32),
            pltpu.VMEM((_RING, _NB, _IW, _JW), jnp.float32),
            pltpu.SemaphoreType.DMA((_RING,)),
            pltpu.SemaphoreType.DMA((_RING,)),
        ],
        compiler_params=pltpu.CompilerParams(
            use_tc_tiling_on_sc=False, needs_layout_passes=False),
    )
    return f(input, white_table)
